# baseline (speedup 1.0000x reference)
import jax
import jax.numpy as jnp
from jax import lax
from jax.experimental import pallas as pl
from jax.experimental.pallas import tpu as pltpu
from jax.experimental.pallas import tpu_sc as plsc

EMB = 64
SEQ = 201
BATCH_ROWS = 1024
NUM_CORES = 2
NUM_SUBCORES = 16
NW = NUM_CORES * NUM_SUBCORES
ROWS_PER_W = BATCH_ROWS // NW
SCALE = 8.0
C0 = 128
C1 = SEQ - C0


def _body(batch_hbm, table_hbm, pos_hbm, out_hbm, idx_v, rows_v, pos_v, sem):
    wid = lax.axis_index("s") * NUM_CORES + lax.axis_index("c")
    pltpu.sync_copy(pos_hbm, pos_v)

    @pl.loop(0, ROWS_PER_W)
    def _row(j):
        b = wid * ROWS_PER_W + j
        pltpu.sync_copy(batch_hbm.at[b], idx_v)
        cp0 = pltpu.async_copy(
            table_hbm.at[idx_v.at[pl.ds(0, C0)]], rows_v.at[pl.ds(0, C0)], sem
        )
        cp1 = pltpu.async_copy(
            table_hbm.at[idx_v.at[pl.ds(C0, C1)]], rows_v.at[pl.ds(C0, C1)], sem
        )
        cp0.wait()
        cp1.wait()

        @pl.loop(0, SEQ)
        def _fma(r):
            for c in range(EMB // 16):
                seg = pl.ds(c * 16, 16)
                rows_v[r, seg] = rows_v[r, seg] * SCALE + pos_v[r, seg]

        pltpu.sync_copy(rows_v, out_hbm.at[b])


_encodings = pl.kernel(
    _body,
    out_type=jax.ShapeDtypeStruct((BATCH_ROWS, SEQ, EMB), jnp.float32),
    mesh=plsc.VectorSubcoreMesh(core_axis_name="c", subcore_axis_name="s"),
    compiler_params=pltpu.CompilerParams(use_tc_tiling_on_sc=False),
    scratch_types=[
        pltpu.VMEM((SEQ,), jnp.int32),
        pltpu.VMEM((SEQ, EMB), jnp.float32),
        pltpu.VMEM((SEQ, EMB), jnp.float32),
        pltpu.SemaphoreType.DMA,
    ],
)


@jax.jit
def kernel(batch, table, pos_table):
    return _encodings(batch.astype(jnp.int32), table, pos_table)
